# initial kernel scaffold (unmeasured)
import jax
import jax.numpy as jnp
from jax import lax
from jax.experimental import pallas as pl
from jax.experimental.pallas import tpu as pltpu

B, SQ, H, D = 8, 8, 16, 128
SCALE = D ** -0.5


def _local_body(q_ref, k_ref, v_ref, o_ref, m_ref, l_ref):
    for h in range(H):
        q = q_ref[0, :, h, :].astype(jnp.bfloat16)
        k = k_ref[0, :, h, :].astype(jnp.bfloat16)
        s = lax.dot_general(
            q, k, (((1,), (1,)), ((), ())),
            preferred_element_type=jnp.float32) * SCALE
        m = jnp.max(s, axis=1, keepdims=True)
        p = jnp.exp(s - m)
        l = jnp.sum(p, axis=1, keepdims=True)
        v = v_ref[0, :, h, :].astype(jnp.bfloat16)
        o = lax.dot_general(
            p.astype(jnp.bfloat16), v, (((1,), (0,)), ((), ())),
            preferred_element_type=jnp.float32)
        o_ref[0, :, h, :] = o
        m_ref[0, :, h:h + 1] = m
        l_ref[0, :, h:h + 1] = l


def _local_partial(Q, K, V):
    skv = K.shape[1]
    return pl.pallas_call(
        _local_body,
        grid=(B,),
        in_specs=[
            pl.BlockSpec((1, SQ, H, D), lambda b: (b, 0, 0, 0)),
            pl.BlockSpec((1, skv, H, D), lambda b: (b, 0, 0, 0)),
            pl.BlockSpec((1, skv, H, D), lambda b: (b, 0, 0, 0)),
        ],
        out_specs=[
            pl.BlockSpec((1, SQ, H, D), lambda b: (b, 0, 0, 0)),
            pl.BlockSpec((1, SQ, H), lambda b: (b, 0, 0)),
            pl.BlockSpec((1, SQ, H), lambda b: (b, 0, 0)),
        ],
        out_shape=[
            jax.ShapeDtypeStruct((B, SQ, H, D), jnp.float32),
            jax.ShapeDtypeStruct((B, SQ, H), jnp.float32),
            jax.ShapeDtypeStruct((B, SQ, H), jnp.float32),
        ],
    )(Q, K, V)


def _combine_body(o_ref, m_ref, l_ref, out_ref,
                  send_o, recv_o, send_ml, recv_ml,
                  send_sem_o, recv_sem_o, send_sem_ml, recv_sem_ml):
    mx = lax.axis_index("x")
    my = lax.axis_index("y")
    mz = lax.axis_index("z")

    barrier = pltpu.get_barrier_semaphore()
    for s in range(2):
        pl.semaphore_signal(
            barrier, inc=1,
            device_id=(mx, my, mz ^ (1 << s)),
            device_id_type=pl.DeviceIdType.MESH)
    pl.semaphore_wait(barrier, 2)

    out_ref[...] = o_ref[...]
    m_acc = m_ref[...]
    l_acc = l_ref[...]

    for s in range(2):
        pz = mz ^ (1 << s)
        send_o[s] = out_ref[...].astype(jnp.bfloat16)
        send_ml[s, 0] = m_acc
        send_ml[s, 1] = l_acc
        rdma_o = pltpu.make_async_remote_copy(
            src_ref=send_o.at[s], dst_ref=recv_o.at[s],
            send_sem=send_sem_o.at[s], recv_sem=recv_sem_o.at[s],
            device_id=(mx, my, pz), device_id_type=pl.DeviceIdType.MESH)
        rdma_ml = pltpu.make_async_remote_copy(
            src_ref=send_ml.at[s], dst_ref=recv_ml.at[s],
            send_sem=send_sem_ml.at[s], recv_sem=recv_sem_ml.at[s],
            device_id=(mx, my, pz), device_id_type=pl.DeviceIdType.MESH)
        rdma_o.start()
        rdma_ml.start()
        rdma_ml.wait()
        rdma_o.wait()

        m_other = recv_ml[s, 0]
        l_other = recv_ml[s, 1]
        m_new = jnp.maximum(m_acc, m_other)
        a_self = jnp.exp(m_acc - m_new)
        a_other = jnp.exp(m_other - m_new)
        out_ref[...] = (out_ref[...] * a_self[..., None]
                        + recv_o[s].astype(jnp.float32) * a_other[..., None])
        l_acc = l_acc * a_self + l_other * a_other
        m_acc = m_new

    out_ref[...] = out_ref[...] / l_acc[..., None]


def _combine(o_part, m_part, l_part):
    return pl.pallas_call(
        _combine_body,
        in_specs=[
            pl.BlockSpec(memory_space=pltpu.VMEM),
            pl.BlockSpec(memory_space=pltpu.VMEM),
            pl.BlockSpec(memory_space=pltpu.VMEM),
        ],
        out_specs=pl.BlockSpec(memory_space=pltpu.VMEM),
        out_shape=jax.ShapeDtypeStruct((B, SQ, H, D), jnp.float32),
        scratch_shapes=[
            pltpu.VMEM((2, B, SQ, H, D), jnp.bfloat16),
            pltpu.VMEM((2, B, SQ, H, D), jnp.bfloat16),
            pltpu.VMEM((2, 2, B, SQ, H), jnp.float32),
            pltpu.VMEM((2, 2, B, SQ, H), jnp.float32),
            pltpu.SemaphoreType.DMA((2,)),
            pltpu.SemaphoreType.DMA((2,)),
            pltpu.SemaphoreType.DMA((2,)),
            pltpu.SemaphoreType.DMA((2,)),
        ],
        compiler_params=pltpu.CompilerParams(collective_id=0),
    )(o_part, m_part, l_part)


def kernel(Q, K, V):
    o_part, m_part, l_part = _local_partial(Q, K, V)
    return _combine(o_part, m_part, l_part)


# baseline (device time: 169746 ns/iter reference)
import jax
import jax.numpy as jnp
from jax import lax
from jax.experimental import pallas as pl
from jax.experimental.pallas import tpu as pltpu

B, SQ, H, D = 8, 8, 16, 128
SCALE = D ** -0.5


def _local_body(q_ref, k_ref, v_ref, o_ref, m_ref, l_ref):
    for h in range(H):
        q = q_ref[0, :, h, :].astype(jnp.bfloat16)
        k = k_ref[0, :, h, :].astype(jnp.bfloat16)
        s = lax.dot_general(
            q, k, (((1,), (1,)), ((), ())),
            preferred_element_type=jnp.float32) * SCALE
        m = jnp.max(s, axis=1, keepdims=True)
        p = jnp.exp(s - m)
        l = jnp.sum(p, axis=1, keepdims=True)
        v = v_ref[0, :, h, :].astype(jnp.bfloat16)
        o = lax.dot_general(
            p.astype(jnp.bfloat16), v, (((1,), (0,)), ((), ())),
            preferred_element_type=jnp.float32)
        o_ref[0, :, h, :] = o
        m_ref[0, :, h:h + 1] = m
        l_ref[0, :, h:h + 1] = l


def _local_partial(Q, K, V):
    skv = K.shape[1]
    return pl.pallas_call(
        _local_body,
        grid=(B,),
        in_specs=[
            pl.BlockSpec((1, SQ, H, D), lambda b: (b, 0, 0, 0)),
            pl.BlockSpec((1, skv, H, D), lambda b: (b, 0, 0, 0)),
            pl.BlockSpec((1, skv, H, D), lambda b: (b, 0, 0, 0)),
        ],
        out_specs=[
            pl.BlockSpec((1, SQ, H, D), lambda b: (b, 0, 0, 0)),
            pl.BlockSpec((1, SQ, H), lambda b: (b, 0, 0)),
            pl.BlockSpec((1, SQ, H), lambda b: (b, 0, 0)),
        ],
        out_shape=[
            jax.ShapeDtypeStruct((B, SQ, H, D), jnp.float32),
            jax.ShapeDtypeStruct((B, SQ, H), jnp.float32),
            jax.ShapeDtypeStruct((B, SQ, H), jnp.float32),
        ],
        compiler_params=pltpu.CompilerParams(
            vmem_limit_bytes=60 * 1024 * 1024),
    )(Q, K, V)


def _combine_body(o_ref, m_ref, l_ref, out_ref,
                  send_o, recv_o, send_ml, recv_ml,
                  send_sem_o, recv_sem_o, send_sem_ml, recv_sem_ml):
    mx = lax.axis_index("x")
    my = lax.axis_index("y")
    mz = lax.axis_index("z")

    barrier = pltpu.get_barrier_semaphore()
    for s in range(2):
        pl.semaphore_signal(
            barrier, inc=1,
            device_id=(mx, my, mz ^ (1 << s)),
            device_id_type=pl.DeviceIdType.MESH)
    pl.semaphore_wait(barrier, 2)

    out_ref[...] = o_ref[...]
    m_acc = m_ref[...]
    l_acc = l_ref[...]

    for s in range(2):
        pz = mz ^ (1 << s)
        send_o[s] = out_ref[...].astype(jnp.bfloat16)
        send_ml[s, 0] = m_acc
        send_ml[s, 1] = l_acc
        rdma_o = pltpu.make_async_remote_copy(
            src_ref=send_o.at[s], dst_ref=recv_o.at[s],
            send_sem=send_sem_o.at[s], recv_sem=recv_sem_o.at[s],
            device_id=(mx, my, pz), device_id_type=pl.DeviceIdType.MESH)
        rdma_ml = pltpu.make_async_remote_copy(
            src_ref=send_ml.at[s], dst_ref=recv_ml.at[s],
            send_sem=send_sem_ml.at[s], recv_sem=recv_sem_ml.at[s],
            device_id=(mx, my, pz), device_id_type=pl.DeviceIdType.MESH)
        rdma_o.start()
        rdma_ml.start()
        rdma_ml.wait()
        rdma_o.wait()

        m_other = recv_ml[s, 0]
        l_other = recv_ml[s, 1]
        m_new = jnp.maximum(m_acc, m_other)
        a_self = jnp.exp(m_acc - m_new)
        a_other = jnp.exp(m_other - m_new)
        out_ref[...] = (out_ref[...] * a_self[..., None]
                        + recv_o[s].astype(jnp.float32) * a_other[..., None])
        l_acc = l_acc * a_self + l_other * a_other
        m_acc = m_new

    out_ref[...] = out_ref[...] / l_acc[..., None]


def _combine(o_part, m_part, l_part):
    return pl.pallas_call(
        _combine_body,
        in_specs=[
            pl.BlockSpec(memory_space=pltpu.VMEM),
            pl.BlockSpec(memory_space=pltpu.VMEM),
            pl.BlockSpec(memory_space=pltpu.VMEM),
        ],
        out_specs=pl.BlockSpec(memory_space=pltpu.VMEM),
        out_shape=jax.ShapeDtypeStruct((B, SQ, H, D), jnp.float32),
        scratch_shapes=[
            pltpu.VMEM((2, B, SQ, H, D), jnp.bfloat16),
            pltpu.VMEM((2, B, SQ, H, D), jnp.bfloat16),
            pltpu.VMEM((2, 2, B, SQ, H), jnp.float32),
            pltpu.VMEM((2, 2, B, SQ, H), jnp.float32),
            pltpu.SemaphoreType.DMA((2,)),
            pltpu.SemaphoreType.DMA((2,)),
            pltpu.SemaphoreType.DMA((2,)),
            pltpu.SemaphoreType.DMA((2,)),
        ],
        compiler_params=pltpu.CompilerParams(collective_id=0),
    )(o_part, m_part, l_part)


def kernel(Q, K, V):
    o_part, m_part, l_part = _local_partial(Q, K, V)
    return _combine(o_part, m_part, l_part)
